# baseline (speedup 1.0000x reference)
import functools

import jax
import jax.numpy as jnp
from jax import lax
from jax.experimental import pallas as pl
from jax.experimental.pallas import tpu as pltpu
from jax.experimental.pallas import tpu_sc as plsc

NC = 2
NS = 16
NW = NC * NS
EB = 128


def _cdiv(a, b):
  return -(-a // b)


def _deg_body(NP, KA, KB, dst_hbm, ew_hbm, out_hbm, dst_v, ew_v, deg_sh):
  c = lax.axis_index("c")
  s = lax.axis_index("s")
  stripe = NP // NS
  row0 = s * stripe
  start = jnp.where(c == 0, s * KA, NS * KA + s * KB)
  kc = jnp.where(c == 0, KA, KB)
  kmax = max(KA, KB)

  zv = jnp.zeros((16,), jnp.float32)
  for g in range(8):
    ew_v[0, pl.ds(16 * g, 16)] = zv
  for i in range(stripe // 128):
    pltpu.sync_copy(ew_v.at[0], deg_sh.at[pl.ds(row0 + i * 128, 128)])
  plsc.subcore_barrier()

  pltpu.sync_copy(dst_hbm.at[pl.ds(start, kmax), :], dst_v)
  pltpu.sync_copy(ew_hbm.at[pl.ds(start, kmax), :], ew_v)

  def step(j, carry):
    pltpu.sync_copy(ew_v.at[j], deg_sh.at[dst_v.at[j]], add=True)
    return carry

  lax.fori_loop(0, kc, step, 0)
  plsc.subcore_barrier()
  pltpu.sync_copy(deg_sh.at[pl.ds(row0, stripe)],
                  out_hbm.at[c, pl.ds(row0, stripe)])


def _make_deg(NP, KA, KB):
  mesh = plsc.VectorSubcoreMesh(core_axis_name="c", subcore_axis_name="s")
  kmax = max(KA, KB)
  return pl.kernel(
      functools.partial(_deg_body, NP, KA, KB),
      out_type=[jax.ShapeDtypeStruct((NC, NP), jnp.float32)],
      mesh=mesh,
      compiler_params=pltpu.CompilerParams(use_tc_tiling_on_sc=False),
      scratch_types=[
          pltpu.VMEM((kmax, EB), jnp.int32),
          pltpu.VMEM((kmax, EB), jnp.float32),
          pltpu.VMEM_SHARED((NP,), jnp.float32),
      ],
  )


def _agg_body(NP, KA, KB, D, u_hbm, src_hbm, dst_hbm, ew_hbm, out_hbm,
              src_v, dst_v, ew_v, rows0, rows1, srows0, srows1, zb, acc_sh,
              gsem0, gsem1, ssem0, ssem1):
  c = lax.axis_index("c")
  s = lax.axis_index("s")
  stripe = NP // NS
  row0 = s * stripe
  nq = D // 32
  start = jnp.where(c == 0, s * KA, NS * KA + s * KB)
  kc = jnp.where(c == 0, KA, KB)
  kmax = max(KA, KB)
  rows = (rows0, rows1)
  srows = (srows0, srows1)
  gsem = (gsem0, gsem1)
  ssem = (ssem0, ssem1)

  zv = jnp.zeros((32,), jnp.bfloat16)
  for i in range(64):
    for q in range(nq):
      zb[i, pl.ds(32 * q, 32)] = zv
  for i in range(stripe // 64):
    pltpu.sync_copy(zb, acc_sh.at[pl.ds(row0 + i * 64, 64), :])
  plsc.subcore_barrier()

  pltpu.sync_copy(src_hbm.at[pl.ds(start, kmax), :], src_v)
  pltpu.sync_copy(dst_hbm.at[pl.ds(start, kmax), :], dst_v)
  pltpu.sync_copy(ew_hbm.at[pl.ds(start, kmax), :], ew_v)

  idx_consts = [jnp.full((16,), l, jnp.int32) for l in range(16)]

  def scale(j, rv, sv):
    def grp(g, carry):
      ewv = ew_v[j, pl.ds(g * 16, 16)]
      for l in range(16):
        e = g * 16 + l
        w = ewv.at[idx_consts[l]].get(mode="promise_in_bounds")
        wb = plsc.pack(w, w, format=plsc.PackFormat.INTERLEAVED)
        for q in range(nq):
          sv[e, pl.ds(32 * q, 32)] = rv[e, pl.ds(32 * q, 32)] * wb
      return carry
    lax.fori_loop(0, 8, grp, 0)

  def g_start(j, b):
    return pltpu.async_copy(u_hbm.at[src_v.at[j]], rows[b], gsem[b])

  def g_wait(j, b):
    pltpu.make_async_copy(u_hbm.at[src_v.at[j]], rows[b], gsem[b]).wait()

  def s_start(j, b):
    return pltpu.async_copy(srows[b], acc_sh.at[dst_v.at[j]], ssem[b],
                            add=True)

  def s_wait(j, b):
    pltpu.make_async_copy(srows[b], acc_sh.at[dst_v.at[j]], ssem[b]).wait()

  g_start(0, 0)
  g_start(1, 1)
  for b in range(2):
    g_wait(b, b)
    scale(b, rows[b], srows[b])
    g_start(b + 2, b)
    s_start(b, b)

  def step(ji, carry):
    for b in range(2):
      j = 2 + 2 * ji + b
      g_wait(j, b)
      s_wait(j - 2, b)
      scale(j, rows[b], srows[b])
      s_start(j, b)

      @pl.when(j + 2 < kc)
      def _():
        g_start(j + 2, b)
    return carry

  lax.fori_loop(0, (kc - 2) // 2, step, 0)
  for b in range(2):
    s_wait(kc - 2 + b, b)

  plsc.subcore_barrier()
  pltpu.sync_copy(acc_sh.at[pl.ds(row0, stripe), :],
                  out_hbm.at[c, pl.ds(row0, stripe), :])


def _make_agg(NP, KA, KB, D):
  mesh = plsc.VectorSubcoreMesh(core_axis_name="c", subcore_axis_name="s")
  kmax = max(KA, KB)
  return pl.kernel(
      functools.partial(_agg_body, NP, KA, KB, D),
      out_type=[jax.ShapeDtypeStruct((NC, NP, D), jnp.bfloat16)],
      mesh=mesh,
      compiler_params=pltpu.CompilerParams(use_tc_tiling_on_sc=False,
                                           needs_layout_passes=False),
      scratch_types=[
          pltpu.VMEM((kmax, EB), jnp.int32),
          pltpu.VMEM((kmax, EB), jnp.int32),
          pltpu.VMEM((kmax, EB), jnp.float32),
          pltpu.VMEM((EB, D), jnp.bfloat16),
          pltpu.VMEM((EB, D), jnp.bfloat16),
          pltpu.VMEM((EB, D), jnp.bfloat16),
          pltpu.VMEM((EB, D), jnp.bfloat16),
          pltpu.VMEM((64, D), jnp.bfloat16),
          pltpu.VMEM_SHARED((NP, D), jnp.bfloat16),
          pltpu.SemaphoreType.DMA,
          pltpu.SemaphoreType.DMA,
          pltpu.SemaphoreType.DMA,
          pltpu.SemaphoreType.DMA,
      ],
  )


def _gru(W, w_ih, w_hh, b_ih, b_hh):
  d = W.shape[1]
  gx = jnp.dot(W, w_ih.T, preferred_element_type=jnp.float32) + b_ih[None, :]
  gh = jnp.dot(W, w_hh.T, preferred_element_type=jnp.float32) + b_hh[None, :]
  xr, xz, xn = gx[:, :d], gx[:, d:2 * d], gx[:, 2 * d:]
  hr, hz, hn = gh[:, :d], gh[:, d:2 * d], gh[:, 2 * d:]
  r = jax.nn.sigmoid(xr + hr)
  z = jax.nn.sigmoid(xz + hz)
  n = jnp.tanh(xn + r * hn)
  return (1.0 - z) * n + z * W


def _evolve_body(W0, wih0, whh0, bih0, bhh0, W1, wih1, whh1, bih1, bhh1,
                 Wa_ref, Wb_ref):
  Wa_ref[...] = _gru(W0[...], wih0[...], whh0[...], bih0[...], bhh0[...])
  Wb_ref[...] = _gru(W1[...], wih1[...], whh1[...], bih1[...], bhh1[...])


def _evolve(W0, wih0, whh0, bih0, bhh0, W1, wih1, whh1, bih1, bhh1):
  D = W0.shape[0]
  return pl.pallas_call(
      _evolve_body,
      out_shape=[jax.ShapeDtypeStruct((D, D), jnp.float32)] * 2,
  )(W0, wih0, whh0, bih0, bhh0, W1, wih1, whh1, bih1, bhh1)


def _prep_body(degp, xp, Wa, dinv_ref, u0_ref, ub0_ref):
  deg = degp[0, :] + degp[1, :] + 1.0
  dinv = lax.rsqrt(jnp.maximum(deg, 1e-12))
  dinv_ref[...] = dinv
  u0 = jnp.dot(xp[...], Wa[...],
               preferred_element_type=jnp.float32) * dinv[:, None]
  u0_ref[...] = u0
  ub0_ref[...] = u0.astype(jnp.bfloat16)


def _prep(degp, xp, Wa, BR):
  NP, D = xp.shape
  grid = (NP // BR,)
  return pl.pallas_call(
      _prep_body,
      grid=grid,
      in_specs=[
          pl.BlockSpec((NC, BR), lambda i: (0, i)),
          pl.BlockSpec((BR, D), lambda i: (i, 0)),
          pl.BlockSpec((D, D), lambda i: (0, 0)),
      ],
      out_specs=[
          pl.BlockSpec((BR,), lambda i: (i,)),
          pl.BlockSpec((BR, D), lambda i: (i, 0)),
          pl.BlockSpec((BR, D), lambda i: (i, 0)),
      ],
      out_shape=[
          jax.ShapeDtypeStruct((NP,), jnp.float32),
          jax.ShapeDtypeStruct((NP, D), jnp.float32),
          jax.ShapeDtypeStruct((NP, D), jnp.bfloat16),
      ],
  )(degp, xp, Wa)


def _mid_body(accp, u0, dinv, l0w, l0b, Wb, u1_ref, ub1_ref):
  dinv_v = dinv[...]
  acc = accp[0].astype(jnp.float32) + accp[1].astype(jnp.float32)
  th = (acc + u0[...]) * dinv_v[:, None]
  g = jnp.maximum(th, 0.0)
  h = jnp.dot(g, l0w[...].T, preferred_element_type=jnp.float32) + l0b[...][None, :]
  u1 = jnp.dot(h, Wb[...],
               preferred_element_type=jnp.float32) * dinv_v[:, None]
  u1_ref[...] = u1
  ub1_ref[...] = u1.astype(jnp.bfloat16)


def _mid(accp, u0, dinv, l0w, l0b, Wb, BR):
  NP, D = u0.shape
  H = l0w.shape[0]
  grid = (NP // BR,)
  return pl.pallas_call(
      _mid_body,
      grid=grid,
      in_specs=[
          pl.BlockSpec((NC, BR, D), lambda i: (0, i, 0)),
          pl.BlockSpec((BR, D), lambda i: (i, 0)),
          pl.BlockSpec((BR,), lambda i: (i,)),
          pl.BlockSpec((H, D), lambda i: (0, 0)),
          pl.BlockSpec((H,), lambda i: (0,)),
          pl.BlockSpec((H, H), lambda i: (0, 0)),
      ],
      out_specs=[
          pl.BlockSpec((BR, H), lambda i: (i, 0)),
          pl.BlockSpec((BR, H), lambda i: (i, 0)),
      ],
      out_shape=[
          jax.ShapeDtypeStruct((NP, H), jnp.float32),
          jax.ShapeDtypeStruct((NP, H), jnp.bfloat16),
      ],
  )(accp, u0, dinv, l0w, l0b, Wb)


def _fin_body(accp, u1, dinv, l1w, l1b, o_ref):
  dinv_v = dinv[...]
  acc = accp[0].astype(jnp.float32) + accp[1].astype(jnp.float32)
  th = (acc + u1[...]) * dinv_v[:, None]
  y = jnp.dot(th, l1w[...].T, preferred_element_type=jnp.float32) + l1b[...][None, :]
  o_ref[...] = jax.nn.sigmoid(y)


def _fin(accp, u1, dinv, l1w, l1b, BR):
  NP, H = u1.shape
  O = l1w.shape[0]
  grid = (NP // BR,)
  return pl.pallas_call(
      _fin_body,
      grid=grid,
      in_specs=[
          pl.BlockSpec((NC, BR, H), lambda i: (0, i, 0)),
          pl.BlockSpec((BR, H), lambda i: (i, 0)),
          pl.BlockSpec((BR,), lambda i: (i,)),
          pl.BlockSpec((O, H), lambda i: (0, 0)),
          pl.BlockSpec((O,), lambda i: (0,)),
      ],
      out_specs=pl.BlockSpec((BR, O), lambda i: (i, 0)),
      out_shape=jax.ShapeDtypeStruct((NP, O), jnp.float32),
  )(accp, u1, dinv, l1w, l1b)


def kernel(x, edge_index, edge_weight, W0, gru0_w_ih, gru0_w_hh, gru0_b_ih,
           gru0_b_hh, lin0_w, lin0_b, W1, gru1_w_ih, gru1_w_hh, gru1_b_ih,
           gru1_b_hh, lin1_w, lin1_b):
  N, D = x.shape
  E = edge_index.shape[1]
  NP = _cdiv(N, NS * 128) * NS * 128

  F0 = 0.40
  TB = _cdiv(E, EB)
  KA = max(4, 2 * _cdiv(int(round(TB * F0)), 2 * NS))
  KB = max(4, 2 * _cdiv(max(TB - NS * KA, 1), 2 * NS))
  kmax = max(KA, KB)
  TBp = NS * (KA + KB) + kmax
  EP = TBp * EB

  src = edge_index[0].astype(jnp.int32)
  dst = edge_index[1].astype(jnp.int32)
  ew = edge_weight.astype(jnp.float32)
  pad = EP - E
  src3 = jnp.concatenate([src, jnp.zeros((pad,), jnp.int32)]).reshape(TBp, EB)
  dst3 = jnp.concatenate([dst, jnp.zeros((pad,), jnp.int32)]).reshape(TBp, EB)
  ew3 = jnp.concatenate([ew, jnp.zeros((pad,), jnp.float32)]).reshape(TBp, EB)
  xp = jnp.concatenate([x, jnp.zeros((NP - N, D), jnp.float32)])

  Wa, Wb = _evolve(W0, gru0_w_ih, gru0_w_hh, gru0_b_ih, gru0_b_hh,
                   W1, gru1_w_ih, gru1_w_hh, gru1_b_ih, gru1_b_hh)

  (degp,) = _make_deg(NP, KA, KB)(dst3, ew3)

  BR = 512
  dinv, u0, ub0 = _prep(degp, xp, Wa, BR)

  agg = _make_agg(NP, KA, KB, D)
  (acc0,) = agg(ub0, src3, dst3, ew3)
  u1, ub1 = _mid(acc0, u0, dinv, lin0_w, lin0_b, Wb, BR)
  (acc1,) = agg(ub1, src3, dst3, ew3)
  out = _fin(acc1, u1, dinv, lin1_w, lin1_b, BR)
  return out[:N]


# speedup vs baseline: 1.0233x; 1.0233x over previous
import functools

import jax
import jax.numpy as jnp
from jax import lax
from jax.experimental import pallas as pl
from jax.experimental.pallas import tpu as pltpu
from jax.experimental.pallas import tpu_sc as plsc

NC = 2
NS = 16
NW = NC * NS
EB = 128


def _cdiv(a, b):
  return -(-a // b)


def _deg_body(NP, KA, KB, dst_hbm, ew_hbm, out_hbm, dst_v, ew_v, deg_sh):
  c = lax.axis_index("c")
  s = lax.axis_index("s")
  stripe = NP // NS
  row0 = s * stripe
  start = jnp.where(c == 0, s * KA, NS * KA + s * KB)
  kc = jnp.where(c == 0, KA, KB)
  kmax = max(KA, KB)

  zv = jnp.zeros((16,), jnp.float32)
  for g in range(8):
    ew_v[0, pl.ds(16 * g, 16)] = zv
  for i in range(stripe // 128):
    pltpu.sync_copy(ew_v.at[0], deg_sh.at[pl.ds(row0 + i * 128, 128)])
  plsc.subcore_barrier()

  pltpu.sync_copy(dst_hbm.at[pl.ds(start, kmax), :], dst_v)
  pltpu.sync_copy(ew_hbm.at[pl.ds(start, kmax), :], ew_v)

  def step(j, carry):
    pltpu.sync_copy(ew_v.at[j], deg_sh.at[dst_v.at[j]], add=True)
    return carry

  lax.fori_loop(0, kc, step, 0)
  plsc.subcore_barrier()
  pltpu.sync_copy(deg_sh.at[pl.ds(row0, stripe)],
                  out_hbm.at[c, pl.ds(row0, stripe)])


def _make_deg(NP, KA, KB):
  mesh = plsc.VectorSubcoreMesh(core_axis_name="c", subcore_axis_name="s")
  kmax = max(KA, KB)
  return pl.kernel(
      functools.partial(_deg_body, NP, KA, KB),
      out_type=[jax.ShapeDtypeStruct((NC, NP), jnp.float32)],
      mesh=mesh,
      compiler_params=pltpu.CompilerParams(use_tc_tiling_on_sc=False),
      scratch_types=[
          pltpu.VMEM((kmax, EB), jnp.int32),
          pltpu.VMEM((kmax, EB), jnp.float32),
          pltpu.VMEM_SHARED((NP,), jnp.float32),
      ],
  )


def _agg_body(NP, KA, KB, D, u_hbm, src_hbm, dst_hbm, ew_hbm, out_hbm,
              src_v, dst_v, ew_v, rows0, rows1, srows0, srows1, zb, acc_sh,
              gsem0, gsem1, ssem0, ssem1):
  c = lax.axis_index("c")
  s = lax.axis_index("s")
  stripe = NP // NS
  row0 = s * stripe
  nq = D // 32
  start = jnp.where(c == 0, s * KA, NS * KA + s * KB)
  kc = jnp.where(c == 0, KA, KB)
  kmax = max(KA, KB)
  rows = (rows0, rows1)
  srows = (srows0, srows1)
  gsem = (gsem0, gsem1)
  ssem = (ssem0, ssem1)

  zv = jnp.zeros((32,), jnp.bfloat16)
  for i in range(64):
    for q in range(nq):
      zb[i, pl.ds(32 * q, 32)] = zv
  for i in range(stripe // 64):
    pltpu.sync_copy(zb, acc_sh.at[pl.ds(row0 + i * 64, 64), :])
  plsc.subcore_barrier()

  pltpu.sync_copy(src_hbm.at[pl.ds(start, kmax), :], src_v)
  pltpu.sync_copy(dst_hbm.at[pl.ds(start, kmax), :], dst_v)
  pltpu.sync_copy(ew_hbm.at[pl.ds(start, kmax), :], ew_v)

  idx_consts = [jnp.full((16,), l, jnp.int32) for l in range(16)]

  def scale(j, rv, sv):
    def grp(g, carry):
      ewv = ew_v[j, pl.ds(g * 16, 16)]
      for l in range(16):
        e = g * 16 + l
        w = ewv.at[idx_consts[l]].get(mode="promise_in_bounds")
        wb = plsc.pack(w, w, format=plsc.PackFormat.INTERLEAVED)
        for q in range(nq):
          sv[e, pl.ds(32 * q, 32)] = rv[e, pl.ds(32 * q, 32)] * wb
      return carry
    lax.fori_loop(0, 8, grp, 0)

  def g_start(j, b):
    return pltpu.async_copy(u_hbm.at[src_v.at[j]], rows[b], gsem[b])

  def g_wait(j, b):
    pltpu.make_async_copy(u_hbm.at[src_v.at[j]], rows[b], gsem[b]).wait()

  def s_start(j, b):
    return pltpu.async_copy(srows[b], acc_sh.at[dst_v.at[j]], ssem[b],
                            add=True)

  def s_wait(j, b):
    pltpu.make_async_copy(srows[b], acc_sh.at[dst_v.at[j]], ssem[b]).wait()

  g_start(0, 0)
  g_start(1, 1)
  for b in range(2):
    g_wait(b, b)
    scale(b, rows[b], srows[b])
    g_start(b + 2, b)
    s_start(b, b)

  def step(ji, carry):
    for b in range(2):
      j = 2 + 2 * ji + b
      g_wait(j, b)
      s_wait(j - 2, b)
      scale(j, rows[b], srows[b])
      s_start(j, b)

      @pl.when(j + 2 < kc)
      def _():
        g_start(j + 2, b)
    return carry

  lax.fori_loop(0, (kc - 2) // 2, step, 0)
  for b in range(2):
    s_wait(kc - 2 + b, b)

  plsc.subcore_barrier()
  pltpu.sync_copy(acc_sh.at[pl.ds(row0, stripe), :],
                  out_hbm.at[c, pl.ds(row0, stripe), :])


def _make_agg(NP, KA, KB, D):
  mesh = plsc.VectorSubcoreMesh(core_axis_name="c", subcore_axis_name="s")
  kmax = max(KA, KB)
  return pl.kernel(
      functools.partial(_agg_body, NP, KA, KB, D),
      out_type=[jax.ShapeDtypeStruct((NC, NP, D), jnp.bfloat16)],
      mesh=mesh,
      compiler_params=pltpu.CompilerParams(use_tc_tiling_on_sc=False,
                                           needs_layout_passes=False),
      scratch_types=[
          pltpu.VMEM((kmax, EB), jnp.int32),
          pltpu.VMEM((kmax, EB), jnp.int32),
          pltpu.VMEM((kmax, EB), jnp.float32),
          pltpu.VMEM((EB, D), jnp.bfloat16),
          pltpu.VMEM((EB, D), jnp.bfloat16),
          pltpu.VMEM((EB, D), jnp.bfloat16),
          pltpu.VMEM((EB, D), jnp.bfloat16),
          pltpu.VMEM((64, D), jnp.bfloat16),
          pltpu.VMEM_SHARED((NP, D), jnp.bfloat16),
          pltpu.SemaphoreType.DMA,
          pltpu.SemaphoreType.DMA,
          pltpu.SemaphoreType.DMA,
          pltpu.SemaphoreType.DMA,
      ],
  )


def _gru(W, w_ih, w_hh, b_ih, b_hh):
  d = W.shape[1]
  gx = jnp.dot(W, w_ih.T, preferred_element_type=jnp.float32) + b_ih[None, :]
  gh = jnp.dot(W, w_hh.T, preferred_element_type=jnp.float32) + b_hh[None, :]
  xr, xz, xn = gx[:, :d], gx[:, d:2 * d], gx[:, 2 * d:]
  hr, hz, hn = gh[:, :d], gh[:, d:2 * d], gh[:, 2 * d:]
  r = jax.nn.sigmoid(xr + hr)
  z = jax.nn.sigmoid(xz + hz)
  n = jnp.tanh(xn + r * hn)
  return (1.0 - z) * n + z * W


def _evolve_body(W0, wih0, whh0, bih0, bhh0, W1, wih1, whh1, bih1, bhh1,
                 Wa_ref, Wb_ref):
  Wa_ref[...] = _gru(W0[...], wih0[...], whh0[...], bih0[...], bhh0[...])
  Wb_ref[...] = _gru(W1[...], wih1[...], whh1[...], bih1[...], bhh1[...])


def _evolve(W0, wih0, whh0, bih0, bhh0, W1, wih1, whh1, bih1, bhh1):
  D = W0.shape[0]
  return pl.pallas_call(
      _evolve_body,
      out_shape=[jax.ShapeDtypeStruct((D, D), jnp.float32)] * 2,
  )(W0, wih0, whh0, bih0, bhh0, W1, wih1, whh1, bih1, bhh1)


def _prep_body(degp, xp, Wa, dinv_ref, u0_ref, ub0_ref):
  deg = degp[0, :] + degp[1, :] + 1.0
  dinv = lax.rsqrt(jnp.maximum(deg, 1e-12))
  dinv_ref[...] = dinv
  u0 = jnp.dot(xp[...], Wa[...],
               preferred_element_type=jnp.float32) * dinv[:, None]
  u0_ref[...] = u0
  ub0_ref[...] = u0.astype(jnp.bfloat16)


def _prep(degp, xp, Wa, BR):
  NP, D = xp.shape
  grid = (NP // BR,)
  return pl.pallas_call(
      _prep_body,
      grid=grid,
      in_specs=[
          pl.BlockSpec((NC, BR), lambda i: (0, i)),
          pl.BlockSpec((BR, D), lambda i: (i, 0)),
          pl.BlockSpec((D, D), lambda i: (0, 0)),
      ],
      out_specs=[
          pl.BlockSpec((BR,), lambda i: (i,)),
          pl.BlockSpec((BR, D), lambda i: (i, 0)),
          pl.BlockSpec((BR, D), lambda i: (i, 0)),
      ],
      out_shape=[
          jax.ShapeDtypeStruct((NP,), jnp.float32),
          jax.ShapeDtypeStruct((NP, D), jnp.float32),
          jax.ShapeDtypeStruct((NP, D), jnp.bfloat16),
      ],
  )(degp, xp, Wa)


def _mid_body(accp, u0, dinv, l0w, l0b, Wb, u1_ref, ub1_ref):
  dinv_v = dinv[...]
  acc = accp[0].astype(jnp.float32) + accp[1].astype(jnp.float32)
  th = (acc + u0[...]) * dinv_v[:, None]
  g = jnp.maximum(th, 0.0)
  h = jnp.dot(g, l0w[...].T, preferred_element_type=jnp.float32) + l0b[...][None, :]
  u1 = jnp.dot(h, Wb[...],
               preferred_element_type=jnp.float32) * dinv_v[:, None]
  u1_ref[...] = u1
  ub1_ref[...] = u1.astype(jnp.bfloat16)


def _mid(accp, u0, dinv, l0w, l0b, Wb, BR):
  NP, D = u0.shape
  H = l0w.shape[0]
  grid = (NP // BR,)
  return pl.pallas_call(
      _mid_body,
      grid=grid,
      in_specs=[
          pl.BlockSpec((NC, BR, D), lambda i: (0, i, 0)),
          pl.BlockSpec((BR, D), lambda i: (i, 0)),
          pl.BlockSpec((BR,), lambda i: (i,)),
          pl.BlockSpec((H, D), lambda i: (0, 0)),
          pl.BlockSpec((H,), lambda i: (0,)),
          pl.BlockSpec((H, H), lambda i: (0, 0)),
      ],
      out_specs=[
          pl.BlockSpec((BR, H), lambda i: (i, 0)),
          pl.BlockSpec((BR, H), lambda i: (i, 0)),
      ],
      out_shape=[
          jax.ShapeDtypeStruct((NP, H), jnp.float32),
          jax.ShapeDtypeStruct((NP, H), jnp.bfloat16),
      ],
  )(accp, u0, dinv, l0w, l0b, Wb)


def _fin_body(accp, u1, dinv, l1w, l1b, o_ref):
  dinv_v = dinv[...]
  acc = accp[0].astype(jnp.float32) + accp[1].astype(jnp.float32)
  th = (acc + u1[...]) * dinv_v[:, None]
  y = jnp.dot(th, l1w[...].T, preferred_element_type=jnp.float32) + l1b[...][None, :]
  o_ref[...] = jax.nn.sigmoid(y)


def _fin(accp, u1, dinv, l1w, l1b, BR):
  NP, H = u1.shape
  O = l1w.shape[0]
  grid = (NP // BR,)
  return pl.pallas_call(
      _fin_body,
      grid=grid,
      in_specs=[
          pl.BlockSpec((NC, BR, H), lambda i: (0, i, 0)),
          pl.BlockSpec((BR, H), lambda i: (i, 0)),
          pl.BlockSpec((BR,), lambda i: (i,)),
          pl.BlockSpec((O, H), lambda i: (0, 0)),
          pl.BlockSpec((O,), lambda i: (0,)),
      ],
      out_specs=pl.BlockSpec((BR, O), lambda i: (i, 0)),
      out_shape=jax.ShapeDtypeStruct((NP, O), jnp.float32),
  )(accp, u1, dinv, l1w, l1b)


def kernel(x, edge_index, edge_weight, W0, gru0_w_ih, gru0_w_hh, gru0_b_ih,
           gru0_b_hh, lin0_w, lin0_b, W1, gru1_w_ih, gru1_w_hh, gru1_b_ih,
           gru1_b_hh, lin1_w, lin1_b):
  N, D = x.shape
  E = edge_index.shape[1]
  NP = _cdiv(N, NS * 128) * NS * 128

  F0 = 0.35
  TB = _cdiv(E, EB)
  KA = max(4, 2 * _cdiv(int(round(TB * F0)), 2 * NS))
  KB = max(4, 2 * _cdiv(max(TB - NS * KA, 1), 2 * NS))
  kmax = max(KA, KB)
  TBp = NS * (KA + KB) + kmax
  EP = TBp * EB

  src = edge_index[0].astype(jnp.int32)
  dst = edge_index[1].astype(jnp.int32)
  ew = edge_weight.astype(jnp.float32)
  pad = EP - E
  src3 = jnp.concatenate([src, jnp.zeros((pad,), jnp.int32)]).reshape(TBp, EB)
  dst3 = jnp.concatenate([dst, jnp.zeros((pad,), jnp.int32)]).reshape(TBp, EB)
  ew3 = jnp.concatenate([ew, jnp.zeros((pad,), jnp.float32)]).reshape(TBp, EB)
  xp = jnp.concatenate([x, jnp.zeros((NP - N, D), jnp.float32)])

  Wa, Wb = _evolve(W0, gru0_w_ih, gru0_w_hh, gru0_b_ih, gru0_b_hh,
                   W1, gru1_w_ih, gru1_w_hh, gru1_b_ih, gru1_b_hh)

  (degp,) = _make_deg(NP, KA, KB)(dst3, ew3)

  BR = 512
  dinv, u0, ub0 = _prep(degp, xp, Wa, BR)

  agg = _make_agg(NP, KA, KB, D)
  (acc0,) = agg(ub0, src3, dst3, ew3)
  u1, ub1 = _mid(acc0, u0, dinv, lin0_w, lin0_b, Wb, BR)
  (acc1,) = agg(ub1, src3, dst3, ew3)
  out = _fin(acc1, u1, dinv, lin1_w, lin1_b, BR)
  return out[:N]


# speedup vs baseline: 1.0254x; 1.0021x over previous
import functools

import jax
import jax.numpy as jnp
from jax import lax
from jax.experimental import pallas as pl
from jax.experimental.pallas import tpu as pltpu
from jax.experimental.pallas import tpu_sc as plsc

NC = 2
NS = 16
NW = NC * NS
EB = 128


def _cdiv(a, b):
  return -(-a // b)


def _deg_body(NP, KA, KB, dst_hbm, ew_hbm, out_hbm, dst_v, ew_v, deg_sh):
  c = lax.axis_index("c")
  s = lax.axis_index("s")
  stripe = NP // NS
  row0 = s * stripe
  start = jnp.where(c == 0, s * KA, NS * KA + s * KB)
  kc = jnp.where(c == 0, KA, KB)
  kmax = max(KA, KB)

  zv = jnp.zeros((16,), jnp.float32)
  for g in range(8):
    ew_v[0, pl.ds(16 * g, 16)] = zv
  for i in range(stripe // 128):
    pltpu.sync_copy(ew_v.at[0], deg_sh.at[pl.ds(row0 + i * 128, 128)])
  plsc.subcore_barrier()

  pltpu.sync_copy(dst_hbm.at[pl.ds(start, kmax), :], dst_v)
  pltpu.sync_copy(ew_hbm.at[pl.ds(start, kmax), :], ew_v)

  def step(j, carry):
    pltpu.sync_copy(ew_v.at[j], deg_sh.at[dst_v.at[j]], add=True)
    return carry

  lax.fori_loop(0, kc, step, 0)
  plsc.subcore_barrier()
  pltpu.sync_copy(deg_sh.at[pl.ds(row0, stripe)],
                  out_hbm.at[c, pl.ds(row0, stripe)])


def _make_deg(NP, KA, KB):
  mesh = plsc.VectorSubcoreMesh(core_axis_name="c", subcore_axis_name="s")
  kmax = max(KA, KB)
  return pl.kernel(
      functools.partial(_deg_body, NP, KA, KB),
      out_type=[jax.ShapeDtypeStruct((NC, NP), jnp.float32)],
      mesh=mesh,
      compiler_params=pltpu.CompilerParams(use_tc_tiling_on_sc=False),
      scratch_types=[
          pltpu.VMEM((kmax, EB), jnp.int32),
          pltpu.VMEM((kmax, EB), jnp.float32),
          pltpu.VMEM_SHARED((NP,), jnp.float32),
      ],
  )


NB = 2


def _agg_body(NP, KA, KB, D, u_hbm, src_hbm, dst_hbm, ew_hbm, out_hbm,
              src_v, dst_v, ew_v, *bufs):
  rows = bufs[:NB]
  srows = bufs[NB:2 * NB]
  zb = bufs[2 * NB]
  acc_sh = bufs[2 * NB + 1]
  gsem = bufs[2 * NB + 2:2 * NB + 2 + NB]
  ssem = bufs[2 * NB + 2 + NB:2 * NB + 2 + 2 * NB]
  c = lax.axis_index("c")
  s = lax.axis_index("s")
  stripe = NP // NS
  row0 = s * stripe
  nq = D // 32
  start = jnp.where(c == 0, s * KA, NS * KA + s * KB)
  kc = jnp.where(c == 0, KA, KB)
  kmax = max(KA, KB)

  zv = jnp.zeros((32,), jnp.bfloat16)
  for i in range(64):
    for q in range(nq):
      zb[i, pl.ds(32 * q, 32)] = zv
  for i in range(stripe // 64):
    pltpu.sync_copy(zb, acc_sh.at[pl.ds(row0 + i * 64, 64), :])
  plsc.subcore_barrier()

  pltpu.sync_copy(src_hbm.at[pl.ds(start, kmax), :], src_v)
  pltpu.sync_copy(dst_hbm.at[pl.ds(start, kmax), :], dst_v)
  pltpu.sync_copy(ew_hbm.at[pl.ds(start, kmax), :], ew_v)

  idx_consts = [jnp.full((16,), l, jnp.int32) for l in range(16)]

  def scale(j, rv, sv):
    def grp(g, carry):
      ewv = ew_v[j, pl.ds(g * 16, 16)]
      for l in range(16):
        e = g * 16 + l
        w = ewv.at[idx_consts[l]].get(mode="promise_in_bounds")
        wb = plsc.pack(w, w, format=plsc.PackFormat.INTERLEAVED)
        for q in range(nq):
          sv[e, pl.ds(32 * q, 32)] = rv[e, pl.ds(32 * q, 32)] * wb
      return carry
    lax.fori_loop(0, 8, grp, 0)

  def g_start(j, b):
    return pltpu.async_copy(u_hbm.at[src_v.at[j]], rows[b], gsem[b])

  def g_wait(j, b):
    pltpu.make_async_copy(u_hbm.at[src_v.at[j]], rows[b], gsem[b]).wait()

  def s_start(j, b):
    return pltpu.async_copy(srows[b], acc_sh.at[dst_v.at[j]], ssem[b],
                            add=True)

  def s_wait(j, b):
    pltpu.make_async_copy(srows[b], acc_sh.at[dst_v.at[j]], ssem[b]).wait()

  for b in range(NB):
    g_start(b, b)
  for b in range(NB):
    g_wait(b, b)
    scale(b, rows[b], srows[b])
    g_start(b + NB, b)
    s_start(b, b)

  def step(ji, carry):
    for b in range(NB):
      j = NB + NB * ji + b
      g_wait(j, b)
      s_wait(j - NB, b)
      scale(j, rows[b], srows[b])
      s_start(j, b)

      @pl.when(j + NB < kc)
      def _():
        g_start(j + NB, b)
    return carry

  lax.fori_loop(0, (kc - NB) // NB, step, 0)
  for b in range(NB):
    s_wait(kc - NB + b, b)

  plsc.subcore_barrier()
  pltpu.sync_copy(acc_sh.at[pl.ds(row0, stripe), :],
                  out_hbm.at[c, pl.ds(row0, stripe), :])


def _make_agg(NP, KA, KB, D):
  mesh = plsc.VectorSubcoreMesh(core_axis_name="c", subcore_axis_name="s")
  kmax = max(KA, KB)
  return pl.kernel(
      functools.partial(_agg_body, NP, KA, KB, D),
      out_type=[jax.ShapeDtypeStruct((NC, NP, D), jnp.bfloat16)],
      mesh=mesh,
      compiler_params=pltpu.CompilerParams(use_tc_tiling_on_sc=False,
                                           needs_layout_passes=False),
      scratch_types=(
          [pltpu.VMEM((kmax, EB), jnp.int32),
           pltpu.VMEM((kmax, EB), jnp.int32),
           pltpu.VMEM((kmax, EB), jnp.float32)]
          + [pltpu.VMEM((EB, D), jnp.bfloat16)] * (2 * NB)
          + [pltpu.VMEM((64, D), jnp.bfloat16),
             pltpu.VMEM_SHARED((NP, D), jnp.bfloat16)]
          + [pltpu.SemaphoreType.DMA] * (2 * NB)
      ),
  )


def _gru(W, w_ih, w_hh, b_ih, b_hh):
  d = W.shape[1]
  gx = jnp.dot(W, w_ih.T, preferred_element_type=jnp.float32) + b_ih[None, :]
  gh = jnp.dot(W, w_hh.T, preferred_element_type=jnp.float32) + b_hh[None, :]
  xr, xz, xn = gx[:, :d], gx[:, d:2 * d], gx[:, 2 * d:]
  hr, hz, hn = gh[:, :d], gh[:, d:2 * d], gh[:, 2 * d:]
  r = jax.nn.sigmoid(xr + hr)
  z = jax.nn.sigmoid(xz + hz)
  n = jnp.tanh(xn + r * hn)
  return (1.0 - z) * n + z * W


def _evolve_body(W0, wih0, whh0, bih0, bhh0, W1, wih1, whh1, bih1, bhh1,
                 Wa_ref, Wb_ref):
  Wa_ref[...] = _gru(W0[...], wih0[...], whh0[...], bih0[...], bhh0[...])
  Wb_ref[...] = _gru(W1[...], wih1[...], whh1[...], bih1[...], bhh1[...])


def _evolve(W0, wih0, whh0, bih0, bhh0, W1, wih1, whh1, bih1, bhh1):
  D = W0.shape[0]
  return pl.pallas_call(
      _evolve_body,
      out_shape=[jax.ShapeDtypeStruct((D, D), jnp.float32)] * 2,
  )(W0, wih0, whh0, bih0, bhh0, W1, wih1, whh1, bih1, bhh1)


def _prep_body(degp, xp, Wa, dinv_ref, u0_ref, ub0_ref):
  deg = degp[0, :] + degp[1, :] + 1.0
  dinv = lax.rsqrt(jnp.maximum(deg, 1e-12))
  dinv_ref[...] = dinv
  u0 = jnp.dot(xp[...], Wa[...],
               preferred_element_type=jnp.float32) * dinv[:, None]
  u0_ref[...] = u0
  ub0_ref[...] = u0.astype(jnp.bfloat16)


def _prep(degp, xp, Wa, BR):
  NP, D = xp.shape
  grid = (NP // BR,)
  return pl.pallas_call(
      _prep_body,
      grid=grid,
      in_specs=[
          pl.BlockSpec((NC, BR), lambda i: (0, i)),
          pl.BlockSpec((BR, D), lambda i: (i, 0)),
          pl.BlockSpec((D, D), lambda i: (0, 0)),
      ],
      out_specs=[
          pl.BlockSpec((BR,), lambda i: (i,)),
          pl.BlockSpec((BR, D), lambda i: (i, 0)),
          pl.BlockSpec((BR, D), lambda i: (i, 0)),
      ],
      out_shape=[
          jax.ShapeDtypeStruct((NP,), jnp.float32),
          jax.ShapeDtypeStruct((NP, D), jnp.float32),
          jax.ShapeDtypeStruct((NP, D), jnp.bfloat16),
      ],
  )(degp, xp, Wa)


def _mid_body(accp, u0, dinv, l0w, l0b, Wb, u1_ref, ub1_ref):
  dinv_v = dinv[...]
  acc = accp[0].astype(jnp.float32) + accp[1].astype(jnp.float32)
  th = (acc + u0[...]) * dinv_v[:, None]
  g = jnp.maximum(th, 0.0)
  h = jnp.dot(g, l0w[...].T, preferred_element_type=jnp.float32) + l0b[...][None, :]
  u1 = jnp.dot(h, Wb[...],
               preferred_element_type=jnp.float32) * dinv_v[:, None]
  u1_ref[...] = u1
  ub1_ref[...] = u1.astype(jnp.bfloat16)


def _mid(accp, u0, dinv, l0w, l0b, Wb, BR):
  NP, D = u0.shape
  H = l0w.shape[0]
  grid = (NP // BR,)
  return pl.pallas_call(
      _mid_body,
      grid=grid,
      in_specs=[
          pl.BlockSpec((NC, BR, D), lambda i: (0, i, 0)),
          pl.BlockSpec((BR, D), lambda i: (i, 0)),
          pl.BlockSpec((BR,), lambda i: (i,)),
          pl.BlockSpec((H, D), lambda i: (0, 0)),
          pl.BlockSpec((H,), lambda i: (0,)),
          pl.BlockSpec((H, H), lambda i: (0, 0)),
      ],
      out_specs=[
          pl.BlockSpec((BR, H), lambda i: (i, 0)),
          pl.BlockSpec((BR, H), lambda i: (i, 0)),
      ],
      out_shape=[
          jax.ShapeDtypeStruct((NP, H), jnp.float32),
          jax.ShapeDtypeStruct((NP, H), jnp.bfloat16),
      ],
  )(accp, u0, dinv, l0w, l0b, Wb)


def _fin_body(accp, u1, dinv, l1w, l1b, o_ref):
  dinv_v = dinv[...]
  acc = accp[0].astype(jnp.float32) + accp[1].astype(jnp.float32)
  th = (acc + u1[...]) * dinv_v[:, None]
  y = jnp.dot(th, l1w[...].T, preferred_element_type=jnp.float32) + l1b[...][None, :]
  o_ref[...] = jax.nn.sigmoid(y)


def _fin(accp, u1, dinv, l1w, l1b, BR):
  NP, H = u1.shape
  O = l1w.shape[0]
  grid = (NP // BR,)
  return pl.pallas_call(
      _fin_body,
      grid=grid,
      in_specs=[
          pl.BlockSpec((NC, BR, H), lambda i: (0, i, 0)),
          pl.BlockSpec((BR, H), lambda i: (i, 0)),
          pl.BlockSpec((BR,), lambda i: (i,)),
          pl.BlockSpec((O, H), lambda i: (0, 0)),
          pl.BlockSpec((O,), lambda i: (0,)),
      ],
      out_specs=pl.BlockSpec((BR, O), lambda i: (i, 0)),
      out_shape=jax.ShapeDtypeStruct((NP, O), jnp.float32),
  )(accp, u1, dinv, l1w, l1b)


def kernel(x, edge_index, edge_weight, W0, gru0_w_ih, gru0_w_hh, gru0_b_ih,
           gru0_b_hh, lin0_w, lin0_b, W1, gru1_w_ih, gru1_w_hh, gru1_b_ih,
           gru1_b_hh, lin1_w, lin1_b):
  N, D = x.shape
  E = edge_index.shape[1]
  NP = _cdiv(N, NS * 128) * NS * 128

  F0 = 0.35
  TB = _cdiv(E, EB)
  KA = max(2 * NB, NB * _cdiv(int(round(TB * F0)), NB * NS))
  KB = max(2 * NB, NB * _cdiv(max(TB - NS * KA, 1), NB * NS))
  kmax = max(KA, KB)
  TBp = NS * (KA + KB) + kmax
  EP = TBp * EB

  src = edge_index[0].astype(jnp.int32)
  dst = edge_index[1].astype(jnp.int32)
  ew = edge_weight.astype(jnp.float32)
  pad = EP - E
  src3 = jnp.concatenate([src, jnp.zeros((pad,), jnp.int32)]).reshape(TBp, EB)
  dst3 = jnp.concatenate([dst, jnp.zeros((pad,), jnp.int32)]).reshape(TBp, EB)
  ew3 = jnp.concatenate([ew, jnp.zeros((pad,), jnp.float32)]).reshape(TBp, EB)
  xp = jnp.concatenate([x, jnp.zeros((NP - N, D), jnp.float32)])

  Wa, Wb = _evolve(W0, gru0_w_ih, gru0_w_hh, gru0_b_ih, gru0_b_hh,
                   W1, gru1_w_ih, gru1_w_hh, gru1_b_ih, gru1_b_hh)

  (degp,) = _make_deg(NP, KA, KB)(dst3, ew3)

  BR = 512
  dinv, u0, ub0 = _prep(degp, xp, Wa, BR)

  agg = _make_agg(NP, KA, KB, D)
  (acc0,) = agg(ub0, src3, dst3, ew3)
  u1, ub1 = _mid(acc0, u0, dinv, lin0_w, lin0_b, Wb, BR)
  (acc1,) = agg(ub1, src3, dst3, ew3)
  out = _fin(acc1, u1, dinv, lin1_w, lin1_b, BR)
  return out[:N]
